# Initial kernel scaffold; baseline (speedup 1.0000x reference)
import jax
import jax.numpy as jnp
from jax.experimental import pallas as pl


def kernel(x, desc_2d, desc_3d, edge_index, node_graph_ids, W1, ln1_g, ln1_b, W2, res2_W, res2_b, ln2_g, ln2_b, Wmu, bmu, Wlv, blv, Wa, ba, Wv, bv, vr_g, vr_b, WU, WV, fln_g, fln_b, Wh1, bh1, bn_g, bn_b, bn_mean, bn_var, Wh2, bh2):
    raise NotImplementedError("write your pallas kernel here")


# speedup vs baseline: 3.5675x; 3.5675x over previous
import functools

import jax
import jax.numpy as jnp
from jax import lax
from jax.experimental import pallas as pl
from jax.experimental.pallas import tpu as pltpu
from jax.experimental.pallas import tpu_sc as plsc

N = 10000
E = 320000
B = 64
NPAD = 10112
STRIPE = NPAD // 16
CH = 80
EPAD = 32 * CH * 128

_MESH = plsc.VectorSubcoreMesh(core_axis_name="c", subcore_axis_name="s")


def _mm(a, b_t):
    return lax.dot_general(
        a, b_t, (((1,), (1,)), ((), ())),
        preferred_element_type=jnp.float32)


def _ln(h, g, b):
    m = h.mean(-1, keepdims=True)
    v = ((h - m) ** 2).mean(-1, keepdims=True)
    return (h - m) / jnp.sqrt(v + 1e-5) * g + b


def _sc_deg(dst_p, ones_hbm, zeros_hbm):
    @functools.partial(
        pl.kernel,
        mesh=_MESH,
        out_type=jax.ShapeDtypeStruct((2, NPAD, 16), jnp.float32),
        compiler_params=pltpu.CompilerParams(use_tc_tiling_on_sc=False),
        scratch_types=[
            pltpu.VMEM((CH, 128), jnp.int32),
            pltpu.VMEM((128, 16), jnp.float32),
            pltpu.VMEM_SHARED((NPAD, 16), jnp.float32),
        ],
    )
    def k(dst_hbm, ones_h, zeros_h, out_hbm, dst_v, ones_v, acc):
        c = lax.axis_index("c")
        s = lax.axis_index("s")
        w = c * 16 + s
        pltpu.sync_copy(dst_hbm.at[w], dst_v)
        pltpu.sync_copy(ones_h, ones_v)
        pltpu.sync_copy(zeros_h, acc.at[pl.ds(s * STRIPE, STRIPE)])
        plsc.subcore_barrier()

        def body(j, carry):
            pltpu.sync_copy(ones_v, acc.at[dst_v.at[j]], add=True)
            return carry

        lax.fori_loop(0, CH, body, 0)
        plsc.subcore_barrier()
        pltpu.sync_copy(acc.at[pl.ds(s * STRIPE, STRIPE)],
                        out_hbm.at[c].at[pl.ds(s * STRIPE, STRIPE)])

    return k(dst_p, ones_hbm, zeros_hbm)


def _sc_agg(h, src_p, dst_p, zeros_hbm, d):
    @functools.partial(
        pl.kernel,
        mesh=_MESH,
        out_type=jax.ShapeDtypeStruct((2, NPAD, d), jnp.float32),
        compiler_params=pltpu.CompilerParams(use_tc_tiling_on_sc=(d == 128)),
        scratch_types=[
            pltpu.VMEM((CH, 128), jnp.int32),
            pltpu.VMEM((CH, 128), jnp.int32),
            pltpu.VMEM((128, d), jnp.float32),
            pltpu.VMEM_SHARED((NPAD, d), jnp.float32),
            pltpu.SemaphoreType.DMA,
        ],
    )
    def k(h_hbm, src_hbm, dst_hbm, zeros_h, out_hbm, src_v, dst_v, rows, acc, sem):
        c = lax.axis_index("c")
        s = lax.axis_index("s")
        w = c * 16 + s
        pltpu.sync_copy(src_hbm.at[w], src_v)
        pltpu.sync_copy(dst_hbm.at[w], dst_v)
        pltpu.sync_copy(zeros_h, acc.at[pl.ds(s * STRIPE, STRIPE)])
        plsc.subcore_barrier()

        def body(j, carry):
            pltpu.async_copy(h_hbm.at[src_v.at[j]], rows, sem).wait()
            pltpu.sync_copy(rows, acc.at[dst_v.at[j]], add=True)
            return carry

        lax.fori_loop(0, CH, body, 0)
        plsc.subcore_barrier()
        pltpu.sync_copy(acc.at[pl.ds(s * STRIPE, STRIPE)],
                        out_hbm.at[c].at[pl.ds(s * STRIPE, STRIPE)])

    return k(h, src_p, dst_p, zeros_hbm)


def _tc1_body(x_ref, w1_ref, degp_ref, ht_ref, norm_ref):
    deg = degp_ref[0, :N, 0:1] + degp_ref[1, :N, 0:1] + 1.0
    norm = lax.rsqrt(jnp.maximum(deg, 1.0))
    norm_ref[...] = norm
    ht_ref[...] = _mm(x_ref[...], w1_ref[...]) * norm


def _tc1(x, W1, degp):
    return pl.pallas_call(
        _tc1_body,
        out_shape=[jax.ShapeDtypeStruct((N, 128), jnp.float32),
                   jax.ShapeDtypeStruct((N, 1), jnp.float32)],
    )(x, W1, degp)


_BLK2 = 2000


def _tc2_body(p_ref, ht1_ref, norm_ref, x_ref, w2_ref, r2w_ref, r2b_ref,
              ln1g_ref, ln1b_ref, ht2_ref, res2_ref):
    ht1 = ht1_ref[...]
    norm = norm_ref[...]
    agg = p_ref[0] + p_ref[1] + ht1
    h1 = jax.nn.relu(_ln(agg * norm + x_ref[...], ln1g_ref[...], ln1b_ref[...]))
    ht2_ref[...] = _mm(h1, w2_ref[...]) * norm
    res2_ref[...] = _mm(h1, r2w_ref[...]) + r2b_ref[...]


def _tc2(p1, ht1, norm, x, W2, res2_W, res2_b, ln1_g, ln1_b):
    nb = N // _BLK2
    row = lambda i: (i, 0)
    full = lambda i: (0, 0)
    return pl.pallas_call(
        _tc2_body,
        grid=(nb,),
        in_specs=[
            pl.BlockSpec((2, _BLK2, 128), lambda i: (0, i, 0)),
            pl.BlockSpec((_BLK2, 128), row),
            pl.BlockSpec((_BLK2, 1), row),
            pl.BlockSpec((_BLK2, 128), row),
            pl.BlockSpec((64, 128), full),
            pl.BlockSpec((64, 128), full),
            pl.BlockSpec((1, 64), full),
            pl.BlockSpec((1, 128), full),
            pl.BlockSpec((1, 128), full),
        ],
        out_specs=[pl.BlockSpec((_BLK2, 64), row),
                   pl.BlockSpec((_BLK2, 64), row)],
        out_shape=[jax.ShapeDtypeStruct((N, 64), jnp.float32),
                   jax.ShapeDtypeStruct((N, 64), jnp.float32)],
    )(p1, ht1, norm, x, W2, res2_W.reshape(64, 128),
      res2_b.reshape(1, 64), ln1_g.reshape(1, 128), ln1_b.reshape(1, 128))


def _tc3_body(p2_ref, ht2_ref, res2_ref, norm_ref, ids_ref, d3_ref,
              ln2g_ref, ln2b_ref, wmu_ref, bmu_ref, wlv_ref, blv_ref,
              wa_ref, ba_ref, wv_ref, bv_ref, vrg_ref, vrb_ref,
              wu_ref, wvv_ref, flng_ref, flnb_ref, wh1_ref, bh1_ref,
              bng_ref, bnb_ref, bnm_ref, bnv_ref, wh2_ref, bh2_ref,
              out_ref):
    agg2 = p2_ref[0, :N, :] + p2_ref[1, :N, :] + ht2_ref[...]
    h2 = jax.nn.relu(_ln(agg2 * norm_ref[...] + res2_ref[...],
                         ln2g_ref[...], ln2b_ref[...]))
    onehot = (lax.broadcasted_iota(jnp.int32, (B, N), 0)
              == ids_ref[...]).astype(jnp.float32)
    counts = onehot.sum(axis=1, keepdims=True)
    hg = lax.dot_general(onehot, h2, (((1,), (0,)), ((), ())),
                         preferred_element_type=jnp.float32,
                         precision=lax.Precision.HIGHEST)
    hg = hg / jnp.maximum(counts, 1.0)
    mu = _mm(hg, wmu_ref[...]) + bmu_ref[...]
    logv = jnp.clip(_mm(hg, wlv_ref[...]) + blv_ref[...], -8.0, 8.0)
    var = jnp.exp(logv) + 1e-6
    sigma = jnp.sqrt(var)
    z = (d3_ref[...] - mu) / (sigma + 1e-6)
    precision = jnp.minimum(1.0 / var, 50.0)
    gate = jax.nn.sigmoid(_mm(hg, wa_ref[...]) + ba_ref[...])
    v3 = gate * precision * z
    v3r = jax.nn.relu(_ln(_mm(v3, wv_ref[...]) + bv_ref[...],
                          vrg_ref[...], vrb_ref[...]))
    fuse = _ln(_mm(hg, wu_ref[...]) * _mm(v3r, wvv_ref[...]),
               flng_ref[...], flnb_ref[...])
    h1h = _mm(fuse, wh1_ref[...]) + bh1_ref[...]
    h1h = (h1h - bnm_ref[...]) / jnp.sqrt(bnv_ref[...] + 1e-5) \
        * bng_ref[...] + bnb_ref[...]
    out_ref[...] = (jax.nn.relu(h1h) * wh2_ref[...]).sum(
        axis=1, keepdims=True) + bh2_ref[0, 0]


def _tc3(p2, ht2, res2, norm, ids, desc_3d, ln2_g, ln2_b, Wmu, bmu, Wlv, blv,
         Wa, ba, Wv, bv, vr_g, vr_b, WU, WV, fln_g, fln_b, Wh1, bh1,
         bn_g, bn_b, bn_mean, bn_var, Wh2, bh2):
    r = lambda a: a.reshape(1, -1)
    return pl.pallas_call(
        _tc3_body,
        out_shape=jax.ShapeDtypeStruct((B, 1), jnp.float32),
    )(p2, ht2, res2, norm, ids.reshape(1, N), desc_3d,
      r(ln2_g), r(ln2_b), Wmu, r(bmu), Wlv, r(blv), Wa, r(ba), Wv, r(bv),
      r(vr_g), r(vr_b), WU, WV, r(fln_g), r(fln_b), Wh1, r(bh1),
      r(bn_g), r(bn_b), r(bn_mean), r(bn_var), Wh2, r(bh2))


def kernel(x, desc_2d, desc_3d, edge_index, node_graph_ids, W1, ln1_g, ln1_b,
           W2, res2_W, res2_b, ln2_g, ln2_b, Wmu, bmu, Wlv, blv, Wa, ba,
           Wv, bv, vr_g, vr_b, WU, WV, fln_g, fln_b, Wh1, bh1, bn_g, bn_b,
           bn_mean, bn_var, Wh2, bh2):
    src, dst = edge_index[0], edge_index[1]
    pad = EPAD - E
    pad_dst = N + (jnp.arange(pad, dtype=jnp.int32) % (NPAD - N))
    src_p = jnp.concatenate([src, jnp.zeros(pad, jnp.int32)]).reshape(32, CH, 128)
    dst_p = jnp.concatenate([dst, pad_dst]).reshape(32, CH, 128)

    ones16 = jnp.ones((128, 16), jnp.float32)
    zeros16 = jnp.zeros((STRIPE, 16), jnp.float32)
    zeros128 = jnp.zeros((STRIPE, 128), jnp.float32)
    zeros64 = jnp.zeros((STRIPE, 64), jnp.float32)

    degp = _sc_deg(dst_p, ones16, zeros16)
    ht1, norm = _tc1(x, W1, degp)
    p1 = _sc_agg(ht1, src_p, dst_p, zeros128, 128)
    ht2, res2 = _tc2(p1, ht1, norm, x, W2, res2_W, res2_b, ln1_g, ln1_b)
    p2 = _sc_agg(ht2, src_p, dst_p, zeros64, 64)
    return _tc3(p2, ht2, res2, norm, node_graph_ids, desc_3d, ln2_g, ln2_b,
                Wmu, bmu, Wlv, blv, Wa, ba, Wv, bv, vr_g, vr_b, WU, WV,
                fln_g, fln_b, Wh1, bh1, bn_g, bn_b, bn_mean, bn_var, Wh2, bh2)


# speedup vs baseline: 3.9932x; 1.1193x over previous
import functools

import jax
import jax.numpy as jnp
from jax import lax
from jax.experimental import pallas as pl
from jax.experimental.pallas import tpu as pltpu
from jax.experimental.pallas import tpu_sc as plsc

N = 10000
E = 320000
B = 64
NPAD = 10112
STRIPE = NPAD // 16
CH = 80
EPAD = 32 * CH * 128
_DEPTH = 2

_MESH = plsc.VectorSubcoreMesh(core_axis_name="c", subcore_axis_name="s")


def _mm(a, b_t):
    return lax.dot_general(
        a, b_t, (((1,), (1,)), ((), ())),
        preferred_element_type=jnp.float32)


def _ln(h, g, b):
    m = h.mean(-1, keepdims=True)
    v = ((h - m) ** 2).mean(-1, keepdims=True)
    return (h - m) / jnp.sqrt(v + 1e-5) * g + b


def _sc_deg(dst_p, ones_hbm, zeros_hbm):
    @functools.partial(
        pl.kernel,
        mesh=_MESH,
        out_type=jax.ShapeDtypeStruct((2, NPAD, 16), jnp.float32),
        compiler_params=pltpu.CompilerParams(use_tc_tiling_on_sc=False),
        scratch_types=[
            pltpu.VMEM((CH, 128), jnp.int32),
            pltpu.VMEM((128, 16), jnp.float32),
            pltpu.VMEM_SHARED((NPAD, 16), jnp.float32),
        ],
    )
    def k(dst_hbm, ones_h, zeros_h, out_hbm, dst_v, ones_v, acc):
        c = lax.axis_index("c")
        s = lax.axis_index("s")
        w = c * 16 + s
        pltpu.sync_copy(dst_hbm.at[w], dst_v)
        pltpu.sync_copy(ones_h, ones_v)
        pltpu.sync_copy(zeros_h, acc.at[pl.ds(s * STRIPE, STRIPE)])
        plsc.subcore_barrier()

        def body(j, carry):
            pltpu.sync_copy(ones_v, acc.at[dst_v.at[j]], add=True)
            return carry

        lax.fori_loop(0, CH, body, 0)
        plsc.subcore_barrier()
        pltpu.sync_copy(acc.at[pl.ds(s * STRIPE, STRIPE)],
                        out_hbm.at[c].at[pl.ds(s * STRIPE, STRIPE)])

    return k(dst_p, ones_hbm, zeros_hbm)


def _sc_agg(h, src_p, dst_p, zeros_hbm, d):
    @functools.partial(
        pl.kernel,
        mesh=_MESH,
        out_type=jax.ShapeDtypeStruct((2, NPAD, d), jnp.float32),
        compiler_params=pltpu.CompilerParams(use_tc_tiling_on_sc=(d == 128)),
        scratch_types=[
            pltpu.VMEM((CH // 2, 128), jnp.int32),
            pltpu.VMEM((CH // 2, 128), jnp.int32),
            pltpu.VMEM((128, d), jnp.float32),
            pltpu.VMEM((128, d), jnp.float32),
            pltpu.VMEM_SHARED((NPAD, d), jnp.float32),
            pltpu.SemaphoreType.DMA,
            pltpu.SemaphoreType.DMA,
        ],
    )
    def k(h_hbm, src_hbm, dst_hbm, zeros_h, out_hbm, src_v, dst_v,
          rows0, rows1, acc, g0, g1):
        rows = (rows0, rows1)
        gsem = (g0, g1)
        c = lax.axis_index("c")
        s = lax.axis_index("s")
        w = c * 16 + s
        half_n = CH // 2
        pltpu.sync_copy(zeros_h, acc.at[pl.ds(s * STRIPE, STRIPE)])
        plsc.subcore_barrier()

        for h_ in range(2):
            pltpu.sync_copy(src_hbm.at[w].at[pl.ds(h_ * half_n, half_n)], src_v)
            pltpu.sync_copy(dst_hbm.at[w].at[pl.ds(h_ * half_n, half_n)], dst_v)
            for k_ in range(_DEPTH):
                pltpu.async_copy(h_hbm.at[src_v.at[k_]], rows[k_], gsem[k_])

            def body(i, carry):
                base = i * _DEPTH
                for k_ in range(_DEPTH):
                    j = base + k_
                    pltpu.make_async_copy(
                        h_hbm.at[src_v.at[j]], rows[k_], gsem[k_]).wait()
                    pltpu.sync_copy(rows[k_], acc.at[dst_v.at[j]], add=True)
                    pltpu.async_copy(
                        h_hbm.at[src_v.at[j + _DEPTH]], rows[k_], gsem[k_])
                return carry

            lax.fori_loop(0, half_n // _DEPTH - 1, body, 0)
            base = half_n - _DEPTH
            for k_ in range(_DEPTH):
                j = base + k_
                pltpu.make_async_copy(
                    h_hbm.at[src_v.at[j]], rows[k_], gsem[k_]).wait()
                pltpu.sync_copy(rows[k_], acc.at[dst_v.at[j]], add=True)
        plsc.subcore_barrier()
        pltpu.sync_copy(acc.at[pl.ds(s * STRIPE, STRIPE)],
                        out_hbm.at[c].at[pl.ds(s * STRIPE, STRIPE)])

    return k(h, src_p, dst_p, zeros_hbm)


def _tc1_body(x_ref, w1_ref, degp_ref, ht_ref, norm_ref):
    deg = degp_ref[0, :N, 0:1] + degp_ref[1, :N, 0:1] + 1.0
    norm = lax.rsqrt(jnp.maximum(deg, 1.0))
    norm_ref[...] = norm
    ht_ref[...] = _mm(x_ref[...], w1_ref[...]) * norm


def _tc1(x, W1, degp):
    return pl.pallas_call(
        _tc1_body,
        out_shape=[jax.ShapeDtypeStruct((N, 128), jnp.float32),
                   jax.ShapeDtypeStruct((N, 1), jnp.float32)],
    )(x, W1, degp)


_BLK2 = 2000


def _tc2_body(p_ref, ht1_ref, norm_ref, x_ref, w2_ref, r2w_ref, r2b_ref,
              ln1g_ref, ln1b_ref, ht2_ref, res2_ref):
    ht1 = ht1_ref[...]
    norm = norm_ref[...]
    agg = p_ref[0] + p_ref[1] + ht1
    h1 = jax.nn.relu(_ln(agg * norm + x_ref[...], ln1g_ref[...], ln1b_ref[...]))
    ht2_ref[...] = _mm(h1, w2_ref[...]) * norm
    res2_ref[...] = _mm(h1, r2w_ref[...]) + r2b_ref[...]


def _tc2(p1, ht1, norm, x, W2, res2_W, res2_b, ln1_g, ln1_b):
    nb = N // _BLK2
    row = lambda i: (i, 0)
    full = lambda i: (0, 0)
    return pl.pallas_call(
        _tc2_body,
        grid=(nb,),
        in_specs=[
            pl.BlockSpec((2, _BLK2, 128), lambda i: (0, i, 0)),
            pl.BlockSpec((_BLK2, 128), row),
            pl.BlockSpec((_BLK2, 1), row),
            pl.BlockSpec((_BLK2, 128), row),
            pl.BlockSpec((64, 128), full),
            pl.BlockSpec((64, 128), full),
            pl.BlockSpec((1, 64), full),
            pl.BlockSpec((1, 128), full),
            pl.BlockSpec((1, 128), full),
        ],
        out_specs=[pl.BlockSpec((_BLK2, 64), row),
                   pl.BlockSpec((_BLK2, 64), row)],
        out_shape=[jax.ShapeDtypeStruct((N, 64), jnp.float32),
                   jax.ShapeDtypeStruct((N, 64), jnp.float32)],
    )(p1, ht1, norm, x, W2, res2_W.reshape(64, 128),
      res2_b.reshape(1, 64), ln1_g.reshape(1, 128), ln1_b.reshape(1, 128))


def _tc3_body(p2_ref, ht2_ref, res2_ref, norm_ref, ids_ref, d3_ref,
              ln2g_ref, ln2b_ref, wmu_ref, bmu_ref, wlv_ref, blv_ref,
              wa_ref, ba_ref, wv_ref, bv_ref, vrg_ref, vrb_ref,
              wu_ref, wvv_ref, flng_ref, flnb_ref, wh1_ref, bh1_ref,
              bng_ref, bnb_ref, bnm_ref, bnv_ref, wh2_ref, bh2_ref,
              out_ref):
    agg2 = p2_ref[0, :N, :] + p2_ref[1, :N, :] + ht2_ref[...]
    h2 = jax.nn.relu(_ln(agg2 * norm_ref[...] + res2_ref[...],
                         ln2g_ref[...], ln2b_ref[...]))
    onehot = (lax.broadcasted_iota(jnp.int32, (B, N), 0)
              == ids_ref[...]).astype(jnp.float32)
    counts = onehot.sum(axis=1, keepdims=True)
    hg = lax.dot_general(onehot, h2, (((1,), (0,)), ((), ())),
                         preferred_element_type=jnp.float32,
                         precision=lax.Precision.HIGHEST)
    hg = hg / jnp.maximum(counts, 1.0)
    mu = _mm(hg, wmu_ref[...]) + bmu_ref[...]
    logv = jnp.clip(_mm(hg, wlv_ref[...]) + blv_ref[...], -8.0, 8.0)
    var = jnp.exp(logv) + 1e-6
    sigma = jnp.sqrt(var)
    z = (d3_ref[...] - mu) / (sigma + 1e-6)
    precision = jnp.minimum(1.0 / var, 50.0)
    gate = jax.nn.sigmoid(_mm(hg, wa_ref[...]) + ba_ref[...])
    v3 = gate * precision * z
    v3r = jax.nn.relu(_ln(_mm(v3, wv_ref[...]) + bv_ref[...],
                          vrg_ref[...], vrb_ref[...]))
    fuse = _ln(_mm(hg, wu_ref[...]) * _mm(v3r, wvv_ref[...]),
               flng_ref[...], flnb_ref[...])
    h1h = _mm(fuse, wh1_ref[...]) + bh1_ref[...]
    h1h = (h1h - bnm_ref[...]) / jnp.sqrt(bnv_ref[...] + 1e-5) \
        * bng_ref[...] + bnb_ref[...]
    out_ref[...] = (jax.nn.relu(h1h) * wh2_ref[...]).sum(
        axis=1, keepdims=True) + bh2_ref[0, 0]


def _tc3(p2, ht2, res2, norm, ids, desc_3d, ln2_g, ln2_b, Wmu, bmu, Wlv, blv,
         Wa, ba, Wv, bv, vr_g, vr_b, WU, WV, fln_g, fln_b, Wh1, bh1,
         bn_g, bn_b, bn_mean, bn_var, Wh2, bh2):
    r = lambda a: a.reshape(1, -1)
    return pl.pallas_call(
        _tc3_body,
        out_shape=jax.ShapeDtypeStruct((B, 1), jnp.float32),
    )(p2, ht2, res2, norm, ids.reshape(1, N), desc_3d,
      r(ln2_g), r(ln2_b), Wmu, r(bmu), Wlv, r(blv), Wa, r(ba), Wv, r(bv),
      r(vr_g), r(vr_b), WU, WV, r(fln_g), r(fln_b), Wh1, r(bh1),
      r(bn_g), r(bn_b), r(bn_mean), r(bn_var), Wh2, r(bh2))


def kernel(x, desc_2d, desc_3d, edge_index, node_graph_ids, W1, ln1_g, ln1_b,
           W2, res2_W, res2_b, ln2_g, ln2_b, Wmu, bmu, Wlv, blv, Wa, ba,
           Wv, bv, vr_g, vr_b, WU, WV, fln_g, fln_b, Wh1, bh1, bn_g, bn_b,
           bn_mean, bn_var, Wh2, bh2):
    src, dst = edge_index[0], edge_index[1]
    pad = EPAD - E
    pad_dst = N + (jnp.arange(pad, dtype=jnp.int32) % (NPAD - N))
    src_p = jnp.concatenate([src, jnp.zeros(pad, jnp.int32)]).reshape(32, CH, 128)
    dst_p = jnp.concatenate([dst, pad_dst]).reshape(32, CH, 128)

    ones16 = jnp.ones((128, 16), jnp.float32)
    zeros16 = jnp.zeros((STRIPE, 16), jnp.float32)
    zeros128 = jnp.zeros((STRIPE, 128), jnp.float32)
    zeros64 = jnp.zeros((STRIPE, 64), jnp.float32)

    degp = _sc_deg(dst_p, ones16, zeros16)
    ht1, norm = _tc1(x, W1, degp)
    p1 = _sc_agg(ht1, src_p, dst_p, zeros128, 128)
    ht2, res2 = _tc2(p1, ht1, norm, x, W2, res2_W, res2_b, ln1_g, ln1_b)
    p2 = _sc_agg(ht2, src_p, dst_p, zeros64, 64)
    return _tc3(p2, ht2, res2, norm, node_graph_ids, desc_3d, ln2_g, ln2_b,
                Wmu, bmu, Wlv, blv, Wa, ba, Wv, bv, vr_g, vr_b, WU, WV,
                fln_g, fln_b, Wh1, bh1, bn_g, bn_b, bn_mean, bn_var, Wh2, bh2)


# speedup vs baseline: 11.2754x; 2.8237x over previous
import functools

import jax
import jax.numpy as jnp
from jax import lax
from jax.experimental import pallas as pl
from jax.experimental.pallas import tpu as pltpu
from jax.experimental.pallas import tpu_sc as plsc

N = 10000
E = 320000
B = 64
NPAD = 10112
STRIPE = NPAD // 16
CH = 80
EPAD = 32 * CH * 128
_DEPTH = 2

_MESH = plsc.VectorSubcoreMesh(core_axis_name="c", subcore_axis_name="s")


def _mm(a, b_t):
    return lax.dot_general(
        a, b_t, (((1,), (1,)), ((), ())),
        preferred_element_type=jnp.float32)


def _ln(h, g, b):
    m = h.mean(-1, keepdims=True)
    v = ((h - m) ** 2).mean(-1, keepdims=True)
    return (h - m) / jnp.sqrt(v + 1e-5) * g + b


def _sc_deg(dst_p, ones_hbm, zeros_hbm):
    @functools.partial(
        pl.kernel,
        mesh=_MESH,
        out_type=jax.ShapeDtypeStruct((2, NPAD, 16), jnp.float32),
        compiler_params=pltpu.CompilerParams(use_tc_tiling_on_sc=False),
        scratch_types=[
            pltpu.VMEM((CH, 128), jnp.int32),
            pltpu.VMEM((128, 16), jnp.float32),
            pltpu.VMEM_SHARED((NPAD, 16), jnp.float32),
        ],
    )
    def k(dst_hbm, ones_h, zeros_h, out_hbm, dst_v, ones_v, acc):
        c = lax.axis_index("c")
        s = lax.axis_index("s")
        w = c * 16 + s
        pltpu.sync_copy(dst_hbm.at[w], dst_v)
        pltpu.sync_copy(ones_h, ones_v)
        pltpu.sync_copy(zeros_h, acc.at[pl.ds(s * STRIPE, STRIPE)])
        plsc.subcore_barrier()

        def body(j, carry):
            pltpu.sync_copy(ones_v, acc.at[dst_v.at[j]], add=True)
            return carry

        lax.fori_loop(0, CH, body, 0)
        plsc.subcore_barrier()
        pltpu.sync_copy(acc.at[pl.ds(s * STRIPE, STRIPE)],
                        out_hbm.at[c].at[pl.ds(s * STRIPE, STRIPE)])

    return k(dst_p, ones_hbm, zeros_hbm)


def _sc_agg(h, src_p, dst_p, zeros_hbm, d):
    @functools.partial(
        pl.kernel,
        mesh=_MESH,
        out_type=jax.ShapeDtypeStruct((2, NPAD, d), jnp.float32),
        compiler_params=pltpu.CompilerParams(use_tc_tiling_on_sc=(d == 128)),
        scratch_types=[
            pltpu.VMEM((CH // 2, 128), jnp.int32),
            pltpu.VMEM((CH // 2, 128), jnp.int32),
            pltpu.VMEM((128, d), jnp.float32),
            pltpu.VMEM((128, d), jnp.float32),
            pltpu.VMEM_SHARED((NPAD, d), jnp.float32),
            pltpu.SemaphoreType.DMA,
            pltpu.SemaphoreType.DMA,
        ],
    )
    def k(h_hbm, src_hbm, dst_hbm, zeros_h, out_hbm, src_v, dst_v,
          rows0, rows1, acc, g0, g1):
        rows = (rows0, rows1)
        gsem = (g0, g1)
        c = lax.axis_index("c")
        s = lax.axis_index("s")
        w = c * 16 + s
        half_n = CH // 2
        pltpu.sync_copy(zeros_h, acc.at[pl.ds(s * STRIPE, STRIPE)])
        plsc.subcore_barrier()

        for h_ in range(2):
            pltpu.sync_copy(src_hbm.at[w].at[pl.ds(h_ * half_n, half_n)], src_v)
            pltpu.sync_copy(dst_hbm.at[w].at[pl.ds(h_ * half_n, half_n)], dst_v)
            for k_ in range(_DEPTH):
                pltpu.async_copy(h_hbm.at[src_v.at[k_]], rows[k_], gsem[k_])

            def body(i, carry):
                base = i * _DEPTH
                for k_ in range(_DEPTH):
                    j = base + k_
                    pltpu.make_async_copy(
                        h_hbm.at[src_v.at[j]], rows[k_], gsem[k_]).wait()
                    pltpu.sync_copy(rows[k_], acc.at[dst_v.at[j]], add=True)
                    pltpu.async_copy(
                        h_hbm.at[src_v.at[j + _DEPTH]], rows[k_], gsem[k_])
                return carry

            lax.fori_loop(0, half_n // _DEPTH - 1, body, 0)
            base = half_n - _DEPTH
            for k_ in range(_DEPTH):
                j = base + k_
                pltpu.make_async_copy(
                    h_hbm.at[src_v.at[j]], rows[k_], gsem[k_]).wait()
                pltpu.sync_copy(rows[k_], acc.at[dst_v.at[j]], add=True)
        plsc.subcore_barrier()
        pltpu.sync_copy(acc.at[pl.ds(s * STRIPE, STRIPE)],
                        out_hbm.at[c].at[pl.ds(s * STRIPE, STRIPE)])

    return k(h, src_p, dst_p, zeros_hbm)


def _tc1_body(x_ref, w1_ref, degp_ref, ht_ref, norm_ref):
    deg = degp_ref[0, :N, 0:1] + degp_ref[1, :N, 0:1] + 1.0
    norm = lax.rsqrt(jnp.maximum(deg, 1.0))
    norm_ref[...] = norm
    ht_ref[...] = _mm(x_ref[...], w1_ref[...]) * norm


def _tc1(x, W1, degp):
    return pl.pallas_call(
        _tc1_body,
        out_shape=[jax.ShapeDtypeStruct((N, 128), jnp.float32),
                   jax.ShapeDtypeStruct((N, 1), jnp.float32)],
    )(x, W1, degp)


_BLK2 = 2000


def _tc2_body(p_ref, ht1_ref, norm_ref, x_ref, w2_ref, r2w_ref, r2b_ref,
              ln1g_ref, ln1b_ref, ht2_ref, res2_ref):
    ht1 = ht1_ref[...]
    norm = norm_ref[...]
    agg = p_ref[0] + p_ref[1] + ht1
    h1 = jax.nn.relu(_ln(agg * norm + x_ref[...], ln1g_ref[...], ln1b_ref[...]))
    ht2_ref[...] = _mm(h1, w2_ref[...]) * norm
    res2_ref[...] = _mm(h1, r2w_ref[...]) + r2b_ref[...]


def _tc2(p1, ht1, norm, x, W2, res2_W, res2_b, ln1_g, ln1_b):
    nb = N // _BLK2
    row = lambda i: (i, 0)
    full = lambda i: (0, 0)
    return pl.pallas_call(
        _tc2_body,
        grid=(nb,),
        in_specs=[
            pl.BlockSpec((2, _BLK2, 128), lambda i: (0, i, 0)),
            pl.BlockSpec((_BLK2, 128), row),
            pl.BlockSpec((_BLK2, 1), row),
            pl.BlockSpec((_BLK2, 128), row),
            pl.BlockSpec((64, 128), full),
            pl.BlockSpec((64, 128), full),
            pl.BlockSpec((1, 64), full),
            pl.BlockSpec((1, 128), full),
            pl.BlockSpec((1, 128), full),
        ],
        out_specs=[pl.BlockSpec((_BLK2, 64), row),
                   pl.BlockSpec((_BLK2, 64), row)],
        out_shape=[jax.ShapeDtypeStruct((N, 64), jnp.float32),
                   jax.ShapeDtypeStruct((N, 64), jnp.float32)],
    )(p1, ht1, norm, x, W2, res2_W.reshape(64, 128),
      res2_b.reshape(1, 64), ln1_g.reshape(1, 128), ln1_b.reshape(1, 128))


def _tc3_body(p2_ref, ht2_ref, res2_ref, norm_ref, ids_ref, d3_ref,
              ln2g_ref, ln2b_ref, wmu_ref, bmu_ref, wlv_ref, blv_ref,
              wa_ref, ba_ref, wv_ref, bv_ref, vrg_ref, vrb_ref,
              wu_ref, wvv_ref, flng_ref, flnb_ref, wh1_ref, bh1_ref,
              bng_ref, bnb_ref, bnm_ref, bnv_ref, wh2_ref, bh2_ref,
              out_ref):
    agg2 = p2_ref[0, :N, :] + p2_ref[1, :N, :] + ht2_ref[...]
    h2 = jax.nn.relu(_ln(agg2 * norm_ref[...] + res2_ref[...],
                         ln2g_ref[...], ln2b_ref[...]))
    onehot = (lax.broadcasted_iota(jnp.int32, (B, N), 0)
              == ids_ref[...]).astype(jnp.float32)
    counts = onehot.sum(axis=1, keepdims=True)
    hg = lax.dot_general(onehot, h2, (((1,), (0,)), ((), ())),
                         preferred_element_type=jnp.float32,
                         precision=lax.Precision.HIGHEST)
    hg = hg / jnp.maximum(counts, 1.0)
    mu = _mm(hg, wmu_ref[...]) + bmu_ref[...]
    logv = jnp.clip(_mm(hg, wlv_ref[...]) + blv_ref[...], -8.0, 8.0)
    var = jnp.exp(logv) + 1e-6
    sigma = jnp.sqrt(var)
    z = (d3_ref[...] - mu) / (sigma + 1e-6)
    precision = jnp.minimum(1.0 / var, 50.0)
    gate = jax.nn.sigmoid(_mm(hg, wa_ref[...]) + ba_ref[...])
    v3 = gate * precision * z
    v3r = jax.nn.relu(_ln(_mm(v3, wv_ref[...]) + bv_ref[...],
                          vrg_ref[...], vrb_ref[...]))
    fuse = _ln(_mm(hg, wu_ref[...]) * _mm(v3r, wvv_ref[...]),
               flng_ref[...], flnb_ref[...])
    h1h = _mm(fuse, wh1_ref[...]) + bh1_ref[...]
    h1h = (h1h - bnm_ref[...]) / jnp.sqrt(bnv_ref[...] + 1e-5) \
        * bng_ref[...] + bnb_ref[...]
    out_ref[...] = (jax.nn.relu(h1h) * wh2_ref[...]).sum(
        axis=1, keepdims=True) + bh2_ref[0, 0]


def _tc3(p2, ht2, res2, norm, ids, desc_3d, ln2_g, ln2_b, Wmu, bmu, Wlv, blv,
         Wa, ba, Wv, bv, vr_g, vr_b, WU, WV, fln_g, fln_b, Wh1, bh1,
         bn_g, bn_b, bn_mean, bn_var, Wh2, bh2):
    r = lambda a: a.reshape(1, -1)
    return pl.pallas_call(
        _tc3_body,
        out_shape=jax.ShapeDtypeStruct((B, 1), jnp.float32),
    )(p2, ht2, res2, norm, ids.reshape(1, N), desc_3d,
      r(ln2_g), r(ln2_b), Wmu, r(bmu), Wlv, r(blv), Wa, r(ba), Wv, r(bv),
      r(vr_g), r(vr_b), WU, WV, r(fln_g), r(fln_b), Wh1, r(bh1),
      r(bn_g), r(bn_b), r(bn_mean), r(bn_var), Wh2, r(bh2))


def kernel(x, desc_2d, desc_3d, edge_index, node_graph_ids, W1, ln1_g, ln1_b,
           W2, res2_W, res2_b, ln2_g, ln2_b, Wmu, bmu, Wlv, blv, Wa, ba,
           Wv, bv, vr_g, vr_b, WU, WV, fln_g, fln_b, Wh1, bh1, bn_g, bn_b,
           bn_mean, bn_var, Wh2, bh2):
    src, dst = edge_index[0], edge_index[1]
    pad = EPAD - E
    pad_dst = N + (jnp.arange(pad, dtype=jnp.int32) % (NPAD - N))
    pad_src = jnp.arange(pad, dtype=jnp.int32) * 131 % N
    src_p = jnp.concatenate([src, pad_src]).reshape(32, CH, 128)
    dst_p = jnp.concatenate([dst, pad_dst]).reshape(32, CH, 128)

    ones16 = jnp.ones((128, 16), jnp.float32)
    zeros16 = jnp.zeros((STRIPE, 16), jnp.float32)
    zeros128 = jnp.zeros((STRIPE, 128), jnp.float32)
    zeros64 = jnp.zeros((STRIPE, 64), jnp.float32)

    degp = _sc_deg(dst_p, ones16, zeros16)
    ht1, norm = _tc1(x, W1, degp)
    p1 = _sc_agg(ht1, src_p, dst_p, zeros128, 128)
    ht2, res2 = _tc2(p1, ht1, norm, x, W2, res2_W, res2_b, ln1_g, ln1_b)
    p2 = _sc_agg(ht2, src_p, dst_p, zeros64, 64)
    return _tc3(p2, ht2, res2, norm, node_graph_ids, desc_3d, ln2_g, ln2_b,
                Wmu, bmu, Wlv, blv, Wa, ba, Wv, bv, vr_g, vr_b, WU, WV,
                fln_g, fln_b, Wh1, bh1, bn_g, bn_b, bn_mean, bn_var, Wh2, bh2)


# speedup vs baseline: 11.7298x; 1.0403x over previous
import functools

import jax
import jax.numpy as jnp
from jax import lax
from jax.experimental import pallas as pl
from jax.experimental.pallas import tpu as pltpu
from jax.experimental.pallas import tpu_sc as plsc

N = 10000
E = 320000
B = 64
NPAD = 10112
STRIPE = NPAD // 16
CH = 80
EPAD = 32 * CH * 128
_DEPTH = 2

_MESH = plsc.VectorSubcoreMesh(core_axis_name="c", subcore_axis_name="s")


def _mm(a, b_t):
    return lax.dot_general(
        a, b_t, (((1,), (1,)), ((), ())),
        preferred_element_type=jnp.float32)


def _ln(h, g, b):
    m = h.mean(-1, keepdims=True)
    v = ((h - m) ** 2).mean(-1, keepdims=True)
    return (h - m) / jnp.sqrt(v + 1e-5) * g + b


def _sc_deg(dst_p, ones_hbm, zeros_hbm):
    @functools.partial(
        pl.kernel,
        mesh=_MESH,
        out_type=jax.ShapeDtypeStruct((2, NPAD, 16), jnp.float32),
        compiler_params=pltpu.CompilerParams(use_tc_tiling_on_sc=False),
        scratch_types=[
            pltpu.VMEM((CH, 128), jnp.int32),
            pltpu.VMEM((128, 16), jnp.float32),
            pltpu.VMEM_SHARED((NPAD, 16), jnp.float32),
        ],
    )
    def k(dst_hbm, ones_h, zeros_h, out_hbm, dst_v, ones_v, acc):
        c = lax.axis_index("c")
        s = lax.axis_index("s")
        w = c * 16 + s
        pltpu.sync_copy(dst_hbm.at[w], dst_v)
        pltpu.sync_copy(ones_h, ones_v)
        pltpu.sync_copy(zeros_h, acc.at[pl.ds(s * STRIPE, STRIPE)])
        plsc.subcore_barrier()

        def body(j, carry):
            pltpu.sync_copy(ones_v, acc.at[dst_v.at[j]], add=True)
            return carry

        lax.fori_loop(0, CH, body, 0)
        plsc.subcore_barrier()
        pltpu.sync_copy(acc.at[pl.ds(s * STRIPE, STRIPE)],
                        out_hbm.at[c].at[pl.ds(s * STRIPE, STRIPE)])

    return k(dst_p, ones_hbm, zeros_hbm)


def _sc_agg(h, src_p, dst_p, zeros_hbm, d, chunk, depth, nstage):
    cht = (EPAD // 32) // chunk
    sblk = cht // nstage

    @functools.partial(
        pl.kernel,
        mesh=_MESH,
        out_type=jax.ShapeDtypeStruct((2, NPAD, d), jnp.float32),
        compiler_params=pltpu.CompilerParams(use_tc_tiling_on_sc=(d == 128)),
        scratch_types=[
            pltpu.VMEM((sblk, chunk), jnp.int32),
            pltpu.VMEM((sblk, chunk), jnp.int32),
            [pltpu.VMEM((chunk, d), jnp.float32) for _ in range(depth)],
            pltpu.VMEM_SHARED((NPAD, d), jnp.float32),
            [pltpu.SemaphoreType.DMA for _ in range(depth)],
        ],
    )
    def k(h_hbm, src_hbm, dst_hbm, zeros_h, out_hbm, src_v, dst_v,
          rows, acc, gsem):
        c = lax.axis_index("c")
        s = lax.axis_index("s")
        w = c * 16 + s
        pltpu.sync_copy(zeros_h, acc.at[pl.ds(s * STRIPE, STRIPE)])
        plsc.subcore_barrier()

        for h_ in range(nstage):
            pltpu.sync_copy(src_hbm.at[w].at[pl.ds(h_ * sblk, sblk)], src_v)
            pltpu.sync_copy(dst_hbm.at[w].at[pl.ds(h_ * sblk, sblk)], dst_v)
            for k_ in range(depth):
                pltpu.async_copy(h_hbm.at[src_v.at[k_]], rows[k_], gsem[k_])

            def body(i, carry):
                base = i * depth
                for k_ in range(depth):
                    j = base + k_
                    pltpu.make_async_copy(
                        h_hbm.at[src_v.at[j]], rows[k_], gsem[k_]).wait()
                    pltpu.sync_copy(rows[k_], acc.at[dst_v.at[j]], add=True)
                    pltpu.async_copy(
                        h_hbm.at[src_v.at[j + depth]], rows[k_], gsem[k_])
                return carry

            lax.fori_loop(0, sblk // depth - 1, body, 0)
            base = sblk - depth
            for k_ in range(depth):
                j = base + k_
                pltpu.make_async_copy(
                    h_hbm.at[src_v.at[j]], rows[k_], gsem[k_]).wait()
                pltpu.sync_copy(rows[k_], acc.at[dst_v.at[j]], add=True)
        plsc.subcore_barrier()
        pltpu.sync_copy(acc.at[pl.ds(s * STRIPE, STRIPE)],
                        out_hbm.at[c].at[pl.ds(s * STRIPE, STRIPE)])

    return k(h, src_p.reshape(32, cht, chunk), dst_p.reshape(32, cht, chunk),
             zeros_hbm)


def _tc1_body(x_ref, w1_ref, degp_ref, ht_ref, norm_ref):
    deg = degp_ref[0, :N, 0:1] + degp_ref[1, :N, 0:1] + 1.0
    norm = lax.rsqrt(jnp.maximum(deg, 1.0))
    norm_ref[...] = norm
    ht_ref[...] = _mm(x_ref[...], w1_ref[...]) * norm


def _tc1(x, W1, degp):
    return pl.pallas_call(
        _tc1_body,
        out_shape=[jax.ShapeDtypeStruct((N, 128), jnp.float32),
                   jax.ShapeDtypeStruct((N, 1), jnp.float32)],
    )(x, W1, degp)


_BLK2 = 2000


def _tc2_body(p_ref, ht1_ref, norm_ref, x_ref, w2_ref, r2w_ref, r2b_ref,
              ln1g_ref, ln1b_ref, ht2_ref, res2_ref):
    ht1 = ht1_ref[...]
    norm = norm_ref[...]
    agg = p_ref[0] + p_ref[1] + ht1
    h1 = jax.nn.relu(_ln(agg * norm + x_ref[...], ln1g_ref[...], ln1b_ref[...]))
    ht2_ref[...] = _mm(h1, w2_ref[...]) * norm
    res2_ref[...] = _mm(h1, r2w_ref[...]) + r2b_ref[...]


def _tc2(p1, ht1, norm, x, W2, res2_W, res2_b, ln1_g, ln1_b):
    nb = N // _BLK2
    row = lambda i: (i, 0)
    full = lambda i: (0, 0)
    return pl.pallas_call(
        _tc2_body,
        grid=(nb,),
        in_specs=[
            pl.BlockSpec((2, _BLK2, 128), lambda i: (0, i, 0)),
            pl.BlockSpec((_BLK2, 128), row),
            pl.BlockSpec((_BLK2, 1), row),
            pl.BlockSpec((_BLK2, 128), row),
            pl.BlockSpec((64, 128), full),
            pl.BlockSpec((64, 128), full),
            pl.BlockSpec((1, 64), full),
            pl.BlockSpec((1, 128), full),
            pl.BlockSpec((1, 128), full),
        ],
        out_specs=[pl.BlockSpec((_BLK2, 64), row),
                   pl.BlockSpec((_BLK2, 64), row)],
        out_shape=[jax.ShapeDtypeStruct((N, 64), jnp.float32),
                   jax.ShapeDtypeStruct((N, 64), jnp.float32)],
    )(p1, ht1, norm, x, W2, res2_W.reshape(64, 128),
      res2_b.reshape(1, 64), ln1_g.reshape(1, 128), ln1_b.reshape(1, 128))


def _tc3_body(p2_ref, ht2_ref, res2_ref, norm_ref, ids_ref, d3_ref,
              ln2g_ref, ln2b_ref, wmu_ref, bmu_ref, wlv_ref, blv_ref,
              wa_ref, ba_ref, wv_ref, bv_ref, vrg_ref, vrb_ref,
              wu_ref, wvv_ref, flng_ref, flnb_ref, wh1_ref, bh1_ref,
              bng_ref, bnb_ref, bnm_ref, bnv_ref, wh2_ref, bh2_ref,
              out_ref):
    agg2 = p2_ref[0, :N, :] + p2_ref[1, :N, :] + ht2_ref[...]
    h2 = jax.nn.relu(_ln(agg2 * norm_ref[...] + res2_ref[...],
                         ln2g_ref[...], ln2b_ref[...]))
    onehot = (lax.broadcasted_iota(jnp.int32, (B, N), 0)
              == ids_ref[...]).astype(jnp.float32)
    counts = onehot.sum(axis=1, keepdims=True)
    hg = lax.dot_general(onehot, h2, (((1,), (0,)), ((), ())),
                         preferred_element_type=jnp.float32,
                         precision=lax.Precision.HIGHEST)
    hg = hg / jnp.maximum(counts, 1.0)
    mu = _mm(hg, wmu_ref[...]) + bmu_ref[...]
    logv = jnp.clip(_mm(hg, wlv_ref[...]) + blv_ref[...], -8.0, 8.0)
    var = jnp.exp(logv) + 1e-6
    sigma = jnp.sqrt(var)
    z = (d3_ref[...] - mu) / (sigma + 1e-6)
    precision = jnp.minimum(1.0 / var, 50.0)
    gate = jax.nn.sigmoid(_mm(hg, wa_ref[...]) + ba_ref[...])
    v3 = gate * precision * z
    v3r = jax.nn.relu(_ln(_mm(v3, wv_ref[...]) + bv_ref[...],
                          vrg_ref[...], vrb_ref[...]))
    fuse = _ln(_mm(hg, wu_ref[...]) * _mm(v3r, wvv_ref[...]),
               flng_ref[...], flnb_ref[...])
    h1h = _mm(fuse, wh1_ref[...]) + bh1_ref[...]
    h1h = (h1h - bnm_ref[...]) / jnp.sqrt(bnv_ref[...] + 1e-5) \
        * bng_ref[...] + bnb_ref[...]
    out_ref[...] = (jax.nn.relu(h1h) * wh2_ref[...]).sum(
        axis=1, keepdims=True) + bh2_ref[0, 0]


def _tc3(p2, ht2, res2, norm, ids, desc_3d, ln2_g, ln2_b, Wmu, bmu, Wlv, blv,
         Wa, ba, Wv, bv, vr_g, vr_b, WU, WV, fln_g, fln_b, Wh1, bh1,
         bn_g, bn_b, bn_mean, bn_var, Wh2, bh2):
    r = lambda a: a.reshape(1, -1)
    return pl.pallas_call(
        _tc3_body,
        out_shape=jax.ShapeDtypeStruct((B, 1), jnp.float32),
    )(p2, ht2, res2, norm, ids.reshape(1, N), desc_3d,
      r(ln2_g), r(ln2_b), Wmu, r(bmu), Wlv, r(blv), Wa, r(ba), Wv, r(bv),
      r(vr_g), r(vr_b), WU, WV, r(fln_g), r(fln_b), Wh1, r(bh1),
      r(bn_g), r(bn_b), r(bn_mean), r(bn_var), Wh2, r(bh2))


def kernel(x, desc_2d, desc_3d, edge_index, node_graph_ids, W1, ln1_g, ln1_b,
           W2, res2_W, res2_b, ln2_g, ln2_b, Wmu, bmu, Wlv, blv, Wa, ba,
           Wv, bv, vr_g, vr_b, WU, WV, fln_g, fln_b, Wh1, bh1, bn_g, bn_b,
           bn_mean, bn_var, Wh2, bh2):
    src, dst = edge_index[0], edge_index[1]
    pad = EPAD - E
    pad_dst = N + (jnp.arange(pad, dtype=jnp.int32) % (NPAD - N))
    pad_src = jnp.arange(pad, dtype=jnp.int32) * 131 % N
    src_f = jnp.concatenate([src, pad_src])
    dst_f = jnp.concatenate([dst, pad_dst])
    dst_p = dst_f.reshape(32, CH, 128)

    ones16 = jnp.ones((128, 16), jnp.float32)
    zeros16 = jnp.zeros((STRIPE, 16), jnp.float32)
    zeros128 = jnp.zeros((STRIPE, 128), jnp.float32)
    zeros64 = jnp.zeros((STRIPE, 64), jnp.float32)

    degp = _sc_deg(dst_p, ones16, zeros16)
    ht1, norm = _tc1(x, W1, degp)
    p1 = _sc_agg(ht1, src_f, dst_f, zeros128, 128, chunk=64, depth=4, nstage=4)
    ht2, res2 = _tc2(p1, ht1, norm, x, W2, res2_W, res2_b, ln1_g, ln1_b)
    p2 = _sc_agg(ht2, src_f, dst_f, zeros64, 64, chunk=128, depth=4, nstage=4)
    return _tc3(p2, ht2, res2, norm, node_graph_ids, desc_3d, ln2_g, ln2_b,
                Wmu, bmu, Wlv, blv, Wa, ba, Wv, bv, vr_g, vr_b, WU, WV,
                fln_g, fln_b, Wh1, bh1, bn_g, bn_b, bn_mean, bn_var, Wh2, bh2)
